# baseline (speedup 1.0000x reference)
import functools

import jax
import jax.numpy as jnp
from jax import lax
from jax.experimental import pallas as pl
from jax.experimental.pallas import tpu as pltpu
from jax.experimental.pallas import tpu_sc as plsc

N = 10000
E = 320000
NC, NS = 2, 16
NW = NC * NS
CH = 80
CC = 128
GP = 512
RPT_A, RPT_B = 624, 640


def _mesh():
    return plsc.VectorSubcoreMesh(
        core_axis_name="c", subcore_axis_name="s", num_cores=NC,
        num_subcores=NS)


def _zero_fill(buf, rows, width):
    def row(i, _):
        for j in range(width // 16):
            buf[i, pl.ds(j * 16, 16)] = jnp.zeros((16,), jnp.float32)
        return 0
    lax.fori_loop(0, rows, row, 0, unroll=False)


def _zero_spmem_rows(zbuf, acc, base_r, nrows):
    zc = zbuf.shape[0]
    for k in range(nrows // zc):
        pltpu.sync_copy(zbuf, acc.at[pl.ds(base_r + k * zc, zc)])
    rem = nrows % zc
    if rem:
        pltpu.sync_copy(zbuf.at[pl.ds(0, rem)],
                        acc.at[pl.ds(base_r + (nrows // zc) * zc, rem)])


def _per_tile_rows(s, fn):
    @pl.when(s < NS - 1)
    def _():
        fn(s * RPT_A, RPT_A)

    @pl.when(s == NS - 1)
    def _():
        fn((NS - 1) * RPT_A, RPT_B)


def _sc_degree(dst_d, dst_t):
    nchunk = E // CC

    @functools.partial(
        pl.kernel,
        out_type=(jax.ShapeDtypeStruct((NC * N,), jnp.float32),
                  jax.ShapeDtypeStruct((NC * N,), jnp.float32)),
        mesh=_mesh(),
        scratch_types=[
            pltpu.VMEM((CC,), jnp.int32),
            pltpu.VMEM((CC,), jnp.float32),
            pltpu.VMEM((640,), jnp.float32),
            pltpu.VMEM_SHARED((N,), jnp.float32),
            pltpu.VMEM_SHARED((N,), jnp.float32),
        ],
    )
    def body(ed_hbm, et_hbm, outd_hbm, outt_hbm, dst_v, ones_v, vbuf,
             accd, acct):
        c = lax.axis_index("c")
        s = lax.axis_index("s")
        w = c * NS + s

        def fill(i, _):
            ones_v[pl.ds(i * 16, 16)] = jnp.full((16,), 1.0, jnp.float32)
            vbuf[pl.ds(i * 16, 16)] = jnp.zeros((16,), jnp.float32)
            vbuf[pl.ds(320 + i * 16, 16)] = jnp.zeros((16,), jnp.float32)
            vbuf[pl.ds(512 + i * 16, 16)] = jnp.zeros((16,), jnp.float32)
            return 0
        lax.fori_loop(0, CC // 16, fill, 0, unroll=False)

        @pl.when(s < NS - 1)
        def _():
            pltpu.sync_copy(vbuf.at[pl.ds(0, 624)],
                            accd.at[pl.ds(s * 624, 624)])
            pltpu.sync_copy(vbuf.at[pl.ds(0, 624)],
                            acct.at[pl.ds(s * 624, 624)])

        @pl.when(s == NS - 1)
        def _():
            pltpu.sync_copy(vbuf, accd.at[pl.ds((NS - 1) * 624, 640)])
            pltpu.sync_copy(vbuf, acct.at[pl.ds((NS - 1) * 624, 640)])
        plsc.subcore_barrier()

        def step(it, _):
            j = w + it * NW

            @pl.when(j < nchunk)
            def _():
                base = j * CC
                pltpu.sync_copy(ed_hbm.at[pl.ds(base, CC)], dst_v)
                pltpu.sync_copy(ones_v, accd.at[dst_v], add=True)
                pltpu.sync_copy(et_hbm.at[pl.ds(base, CC)], dst_v)
                pltpu.sync_copy(ones_v, acct.at[dst_v], add=True)
            return 0
        lax.fori_loop(0, (nchunk + NW - 1) // NW, step, 0, unroll=False)
        plsc.subcore_barrier()

        for acc, out_hbm in ((accd, outd_hbm), (acct, outt_hbm)):
            @pl.when(s < NS - 1)
            def _():
                pltpu.sync_copy(acc.at[pl.ds(s * 624, 624)],
                                vbuf.at[pl.ds(0, 624)])
                pltpu.sync_copy(vbuf.at[pl.ds(0, 624)],
                                out_hbm.at[pl.ds(c * N + s * 624, 624)])

            @pl.when(s == NS - 1)
            def _():
                pltpu.sync_copy(acc.at[pl.ds((NS - 1) * 624, 640)], vbuf)
                pltpu.sync_copy(
                    vbuf, out_hbm.at[pl.ds(c * N + (NS - 1) * 624, 640)])

    return body(dst_d, dst_t)


def _sc_conv(xs, src, dst):
    ew = E // NW
    nch = ew // CH

    @functools.partial(
        pl.kernel,
        out_type=jax.ShapeDtypeStruct((NC, N, 128), jnp.float32),
        mesh=_mesh(),
        scratch_types=[
            pltpu.VMEM((CH,), jnp.int32),
            pltpu.VMEM((CH,), jnp.int32),
            pltpu.VMEM((CH,), jnp.int32),
            pltpu.VMEM((CH,), jnp.int32),
            pltpu.VMEM((CH, 128), jnp.float32),
            pltpu.VMEM((CH, 128), jnp.float32),
            pltpu.VMEM_SHARED((N, 128), jnp.float32),
            pltpu.SemaphoreType.DMA,
            pltpu.SemaphoreType.DMA,
            pltpu.SemaphoreType.DMA,
            pltpu.SemaphoreType.DMA,
        ],
    )
    def body(xs_hbm, src_hbm, dst_hbm, out_hbm, s0v, d0v, s1v, d1v, r0, r1,
             acc, g0, g1, q0, q1):
        c = lax.axis_index("c")
        s = lax.axis_index("s")
        e0 = (c * NS + s) * ew
        bufs = ((s0v, d0v, r0, g0, q0), (s1v, d1v, r1, g1, q1))
        _zero_fill(r0, CH, 128)
        _per_tile_rows(s, lambda st, nr: _zero_spmem_rows(r0, acc, st, nr))
        plsc.subcore_barrier()

        def chunk(j, sv, dv, rv, gs, qs):
            base = e0 + j * CH
            pltpu.sync_copy(src_hbm.at[pl.ds(base, CH)], sv)
            pltpu.sync_copy(dst_hbm.at[pl.ds(base, CH)], dv)
            pltpu.async_copy(xs_hbm.at[sv], rv, gs).wait()
            pltpu.async_copy(rv, acc.at[dv], qs, add=True)

        def pair(i, _):
            for b, (sv, dv, rv, gs, qs) in enumerate(bufs):
                @pl.when(i >= 1)
                def _():
                    pltpu.make_async_copy(rv, acc.at[dv], qs).wait()
                chunk(2 * i + b, sv, dv, rv, gs, qs)
            return 0
        lax.fori_loop(0, (nch - 1) // 2, pair, 0, unroll=False)
        pltpu.make_async_copy(r0, acc.at[d0v], q0).wait()
        chunk(nch - 1, s0v, d0v, r0, g0, q0)
        pltpu.make_async_copy(r0, acc.at[d0v], q0).wait()
        pltpu.make_async_copy(r1, acc.at[d1v], q1).wait()
        plsc.subcore_barrier()

        def wb(start, nrows):
            pltpu.sync_copy(acc.at[pl.ds(start, nrows)],
                            out_hbm.at[c, pl.ds(start, nrows)])
        _per_tile_rows(s, wb)

    return body(xs, src, dst)


def _sc_conv_cat(xs2, src, dst):
    ew = E // NS
    nch = ew // CH

    @functools.partial(
        pl.kernel,
        out_type=jax.ShapeDtypeStruct((NC, N, 128), jnp.float32),
        mesh=_mesh(),
        scratch_types=[
            pltpu.VMEM((CH,), jnp.int32),
            pltpu.VMEM((CH,), jnp.int32),
            pltpu.VMEM((CH,), jnp.int32),
            pltpu.VMEM((CH,), jnp.int32),
            pltpu.VMEM((CH, 128), jnp.float32),
            pltpu.VMEM((CH, 128), jnp.float32),
            pltpu.VMEM_SHARED((N, 128), jnp.float32),
            pltpu.SemaphoreType.DMA,
            pltpu.SemaphoreType.DMA,
            pltpu.SemaphoreType.DMA,
            pltpu.SemaphoreType.DMA,
        ],
    )
    def body(xs_hbm, src_hbm, dst_hbm, out_hbm, s0v, d0v, s1v, d1v, r0, r1,
             acc, g0, g1, q0, q1):
        c = lax.axis_index("c")
        s = lax.axis_index("s")
        e0 = s * ew
        off = c * N
        bufs = ((s0v, d0v, r0, g0, q0), (s1v, d1v, r1, g1, q1))
        _zero_fill(r0, CH, 128)
        _per_tile_rows(s, lambda st, nr: _zero_spmem_rows(r0, acc, st, nr))
        plsc.subcore_barrier()

        def chunk(j, sv, dv, rv, gs, qs):
            base = e0 + j * CH
            pltpu.sync_copy(src_hbm.at[pl.ds(base, CH)], sv)
            pltpu.sync_copy(dst_hbm.at[pl.ds(base, CH)], dv)
            for k in range(CH // 16):
                sl = pl.ds(k * 16, 16)
                sv[sl] = sv[sl] + off
            pltpu.async_copy(xs_hbm.at[sv], rv, gs).wait()
            pltpu.async_copy(rv, acc.at[dv], qs, add=True)

        def pair(i, _):
            for b, (sv, dv, rv, gs, qs) in enumerate(bufs):
                @pl.when(i >= 1)
                def _():
                    pltpu.make_async_copy(rv, acc.at[dv], qs).wait()
                chunk(2 * i + b, sv, dv, rv, gs, qs)
            return 0
        lax.fori_loop(0, nch // 2, pair, 0, unroll=False)
        pltpu.make_async_copy(r0, acc.at[d0v], q0).wait()
        pltpu.make_async_copy(r1, acc.at[d1v], q1).wait()
        plsc.subcore_barrier()

        def wb(start, nrows):
            pltpu.sync_copy(acc.at[pl.ds(start, nrows)],
                            out_hbm.at[c, pl.ds(start, nrows)])
        _per_tile_rows(s, wb)

    return body(xs2, src, dst)


def _sc_pool(xd, xt, batch_d, batch_t):
    rows_g = GP // NS
    o128 = jax.ShapeDtypeStruct((NC, GP, 128), jnp.float32)

    @functools.partial(
        pl.kernel,
        out_type=(o128,) * 7,
        mesh=_mesh(),
        scratch_types=[
            pltpu.VMEM((CH,), jnp.int32),
            pltpu.VMEM((CH, 128), jnp.float32),
            pltpu.VMEM((CH, 128), jnp.float32),
            pltpu.VMEM((CH, 128), jnp.float32),
            pltpu.VMEM((CH, 128), jnp.float32),
            pltpu.VMEM_SHARED((GP, 128), jnp.float32),
            pltpu.VMEM_SHARED((GP, 128), jnp.float32),
            pltpu.VMEM_SHARED((GP, 128), jnp.float32),
            pltpu.VMEM_SHARED((GP, 128), jnp.float32),
            pltpu.VMEM_SHARED((GP, 128), jnp.float32),
            pltpu.VMEM_SHARED((GP, 128), jnp.float32),
            pltpu.VMEM_SHARED((GP, 128), jnp.float32),
        ],
    )
    def body(xd_hbm, xt_hbm, bd_hbm, bt_hbm,
             pd0_h, pd1_h, pd2_h, pt0_h, pt1_h, cd_h, ct_h,
             bidx, ra, rb, rc, ones_v,
             ad0, ad1, ad2, at0, at1, cntd, cntt):
        c = lax.axis_index("c")
        s = lax.axis_index("s")
        w = c * NS + s
        base_g = s * rows_g
        _zero_fill(ra, rows_g, 128)
        for accr in (ad0, ad1, ad2, at0, at1, cntd, cntt):
            pltpu.sync_copy(ra.at[pl.ds(0, rows_g)],
                            accr.at[pl.ds(base_g, rows_g)])

        def fill1(i, _):
            for j in range(8):
                ones_v[i, pl.ds(j * 16, 16)] = jnp.full((16,), 1.0,
                                                        jnp.float32)
            return 0
        lax.fori_loop(0, CH, fill1, 0, unroll=False)
        plsc.subcore_barrier()

        nchunk = N // CH

        def step(it, _):
            j = w + it * NW

            @pl.when(j < nchunk)
            def _():
                base = j * CH
                pltpu.sync_copy(bd_hbm.at[pl.ds(base, CH)], bidx)
                for k, (rbuf, accr) in enumerate(
                        ((ra, ad0), (rb, ad1), (rc, ad2))):
                    pltpu.sync_copy(
                        xd_hbm.at[pl.ds(base, CH), pl.ds(k * 128, 128)],
                        rbuf)
                    pltpu.sync_copy(rbuf, accr.at[bidx], add=True)
                pltpu.sync_copy(ones_v, cntd.at[bidx], add=True)
                pltpu.sync_copy(bt_hbm.at[pl.ds(base, CH)], bidx)
                for k, (rbuf, accr) in enumerate(((ra, at0), (rb, at1))):
                    pltpu.sync_copy(
                        xt_hbm.at[pl.ds(base, CH), pl.ds(k * 128, 128)],
                        rbuf)
                    pltpu.sync_copy(rbuf, accr.at[bidx], add=True)
                pltpu.sync_copy(ones_v, cntt.at[bidx], add=True)
            return 0
        lax.fori_loop(0, (nchunk + NW - 1) // NW, step, 0, unroll=False)
        plsc.subcore_barrier()
        for accr, out_h in ((ad0, pd0_h), (ad1, pd1_h), (ad2, pd2_h),
                            (at0, pt0_h), (at1, pt1_h), (cntd, cd_h),
                            (cntt, ct_h)):
            pltpu.sync_copy(accr.at[pl.ds(base_g, rows_g)],
                            out_h.at[c, pl.ds(base_g, rows_g)])

    return body(xd, xt, batch_d, batch_t)


_BR = 1000


def _tc_prep(cnt_d, cnt_t, xd, xt):
    def body(cd_r, ct_r, xd_r, xt_r, dd_r, dt_r, od_r, ot_r):
        degd = jnp.sum(cd_r[0], axis=1)[:, None] + 1.0
        dd = lax.rsqrt(jnp.broadcast_to(degd, (_BR, 8)))
        dd_r[...] = dd
        od_r[...] = xd_r[...] * dd[:, 0:1]
        degt = jnp.sum(ct_r[0], axis=1)[:, None] + 1.0
        dt = lax.rsqrt(jnp.broadcast_to(degt, (_BR, 8)))
        dt_r[...] = dt
        ot_r[...] = xt_r[...] * dt[:, 0:1]

    grid = N // _BR
    return pl.pallas_call(
        body,
        grid=(grid,),
        in_specs=[
            pl.BlockSpec((1, _BR, NC), lambda i: (i, 0, 0)),
            pl.BlockSpec((1, _BR, NC), lambda i: (i, 0, 0)),
            pl.BlockSpec((_BR, 128), lambda i: (i, 0)),
            pl.BlockSpec((_BR, 128), lambda i: (i, 0)),
        ],
        out_specs=[
            pl.BlockSpec((_BR, 8), lambda i: (i, 0)),
            pl.BlockSpec((_BR, 8), lambda i: (i, 0)),
            pl.BlockSpec((_BR, 128), lambda i: (i, 0)),
            pl.BlockSpec((_BR, 128), lambda i: (i, 0)),
        ],
        out_shape=[
            jax.ShapeDtypeStruct((N, 8), jnp.float32),
            jax.ShapeDtypeStruct((N, 8), jnp.float32),
            jax.ShapeDtypeStruct((N, 128), jnp.float32),
            jax.ShapeDtypeStruct((N, 128), jnp.float32),
        ],
    )(cnt_d, cnt_t, xd, xt)


def _mm(a, b):
    return lax.dot_general(a, b, (((1,), (0,)), ((), ())),
                           precision=lax.Precision.HIGHEST,
                           preferred_element_type=jnp.float32)


def _tc_layer(acc, xs, dis, wp, bp, scale_end, cat_in=False,
              halves_out=False):
    win, wout = wp.shape

    def body(acc_r, xs_r, dis_r, w_r, b_r, out_r):
        if cat_in:
            a = jnp.concatenate([acc_r[0] + xs_r[0], acc_r[1] + xs_r[1]],
                                axis=1)
        else:
            a = acc_r[0] + acc_r[1] + xs_r[...]
        a = a * dis_r[:, 0:1]
        h = _mm(a, w_r[...])
        h = jnp.maximum(h + b_r[...], 0.0)
        if scale_end:
            h = h * dis_r[:, 0:1]
        if halves_out:
            out_r[0] = h[:, 0:128]
            out_r[1] = h[:, 128:256]
        else:
            out_r[...] = h

    grid = N // _BR
    if cat_in:
        in0 = pl.BlockSpec((NC, _BR, 128), lambda i: (0, i, 0))
        in1 = pl.BlockSpec((NC, _BR, 128), lambda i: (0, i, 0))
    else:
        in0 = pl.BlockSpec((NC, _BR, win), lambda i: (0, i, 0))
        in1 = pl.BlockSpec((_BR, win), lambda i: (i, 0))
    if halves_out:
        out_spec = pl.BlockSpec((NC, _BR, 128), lambda i: (0, i, 0))
        out_shape = jax.ShapeDtypeStruct((NC, N, 128), jnp.float32)
    else:
        out_spec = pl.BlockSpec((_BR, wout), lambda i: (i, 0))
        out_shape = jax.ShapeDtypeStruct((N, wout), jnp.float32)
    return pl.pallas_call(
        body,
        grid=(grid,),
        in_specs=[
            in0, in1,
            pl.BlockSpec((_BR, 8), lambda i: (i, 0)),
            pl.BlockSpec((win, wout), lambda i: (0, 0)),
            pl.BlockSpec((1, wout), lambda i: (0, 0)),
        ],
        out_specs=out_spec,
        out_shape=out_shape,
    )(acc, xs, dis, wp, bp)


def _tc_tail(pools, n1, n2, adjp, weights):
    def body(pd0_r, pd1_r, pd2_r, pt0_r, pt1_r, cd_r, ct_r, n1_r, n2_r,
             adj_r,
             dow1_r, dob1_r, dow2_r, dob2_r, tow1_r, tob1_r, tow2_r, tob2_r,
             agw_r, agb_r, dtw1_r, dtb1_r, dtw2_r, dtb2_r,
             ttw1_r, ttb1_r, ttw2_r, ttb2_r, demb_r, temb_r):
        iota = lax.broadcasted_iota(jnp.int32, (GP, 1), 0)
        psum_d = jnp.concatenate(
            [pd0_r[0] + pd0_r[1], pd1_r[0] + pd1_r[1], pd2_r[0] + pd2_r[1]],
            axis=1)
        psum_t = jnp.concatenate(
            [pt0_r[0] + pt0_r[1], pt1_r[0] + pt1_r[1]], axis=1)
        dge = psum_d / jnp.maximum((cd_r[0] + cd_r[1])[:, 0:1], 1.0)
        dge = jnp.where(iota < n1_r[0, 0], dge, 0.0)
        tge = psum_t / jnp.maximum((ct_r[0] + ct_r[1])[:, 0:1], 1.0)
        tge = jnp.where(iota < n2_r[0, 0], tge, 0.0)
        dout = _mm(jnp.maximum(_mm(dge, dow1_r[...]) + dob1_r[...], 0.0),
                   dow2_r[...]) + dob2_r[...]
        tout = _mm(jnp.maximum(_mm(tge, tow1_r[...]) + tob1_r[...], 0.0),
                   tow2_r[...]) + tob2_r[...]
        feat = jnp.concatenate([dout, tout], axis=0)
        deg = jnp.clip(jnp.sum(adj_r[...], axis=1, keepdims=True), 1.0, None)
        disa = lax.rsqrt(deg)
        h = _mm(feat, agw_r[...])
        aff = jnp.maximum(disa * _mm(adj_r[...], disa * h) + agb_r[...], 0.0)
        dtr = _mm(jnp.maximum(_mm(aff[0:GP], dtw1_r[...]) + dtb1_r[...],
                              0.0), dtw2_r[...]) + dtb2_r[...]
        ttr = _mm(jnp.maximum(
            _mm(aff[GP:2 * GP], ttw1_r[...]) + ttb1_r[...], 0.0),
            ttw2_r[...]) + ttb2_r[...]
        demb_r[:, 0:256] = dout
        demb_r[:, 256:384] = dtr
        temb_r[:, 0:256] = tout
        temb_r[:, 256:384] = ttr

    return pl.pallas_call(
        body,
        out_shape=[
            jax.ShapeDtypeStruct((GP, 384), jnp.float32),
            jax.ShapeDtypeStruct((GP, 384), jnp.float32),
        ],
    )(*pools, n1, n2, adjp, *weights)


def _pad2(a, r, c):
    return jnp.pad(a, ((0, r - a.shape[0]), (0, c - a.shape[1])))


def _padb(b, n):
    return jnp.pad(b, (0, n - b.shape[0])).reshape(1, n)


def kernel(x_drug, edge_index_drug, batch_drug, x_target, edge_index_target,
           batch_target, adj, num_node1s, num_node2s, dg_W1, dg_b1, dg_W2,
           dg_b2, dg_W3, dg_b3, tg_W1, tg_b1, tg_W2, tg_b2, tg_W3, tg_b3,
           do_W1, do_b1, do_W2, do_b2, to_W1, to_b1, to_W2, to_b2, ag_W,
           ag_b, dt_W1, dt_b1, dt_W2, dt_b2, tt_W1, tt_b1, tt_W2, tt_b2):
    f32 = jnp.float32
    src_d = edge_index_drug[0].astype(jnp.int32)
    dst_d = edge_index_drug[1].astype(jnp.int32)
    src_t = edge_index_target[0].astype(jnp.int32)
    dst_t = edge_index_target[1].astype(jnp.int32)
    bat_d = batch_drug.astype(jnp.int32)
    bat_t = batch_target.astype(jnp.int32)

    xd = _pad2(x_drug.astype(f32), N, 128)
    xt = _pad2(x_target.astype(f32), N, 128)

    cnt_d, cnt_t = _sc_degree(dst_d, dst_t)
    cnt_d = cnt_d.reshape(NC, N // _BR, _BR).transpose(1, 2, 0)
    cnt_t = cnt_t.reshape(NC, N // _BR, _BR).transpose(1, 2, 0)
    dis_d, dis_t, xs_d, xs_t = _tc_prep(cnt_d, cnt_t, xd, xt)

    acc = _sc_conv(xs_d, src_d, dst_d)
    xs_d = _tc_layer(acc, xs_d, dis_d, _pad2(dg_W1, 128, 128),
                     _padb(dg_b1, 128), True)
    acc = _sc_conv(xs_d, src_d, dst_d)
    xs3_d = _tc_layer(acc, xs_d, dis_d, _pad2(dg_W2, 128, 256),
                      _padb(dg_b2, 256), True, halves_out=True)
    acc = _sc_conv_cat(xs3_d.reshape(NC * N, 128), src_d, dst_d)
    xfin_d = _tc_layer(acc, xs3_d, dis_d, _pad2(dg_W3, 256, 384),
                       _padb(dg_b3, 384), False, cat_in=True)

    acc = _sc_conv(xs_t, src_t, dst_t)
    xs_t = _tc_layer(acc, xs_t, dis_t, _pad2(tg_W1, 128, 128),
                     _padb(tg_b1, 128), True)
    acc = _sc_conv(xs_t, src_t, dst_t)
    xs_t = _tc_layer(acc, xs_t, dis_t, _pad2(tg_W2, 128, 128),
                     _padb(tg_b2, 128), True)
    acc = _sc_conv(xs_t, src_t, dst_t)
    xfin_t = _tc_layer(acc, xs_t, dis_t, _pad2(tg_W3, 128, 256),
                       _padb(tg_b3, 256), False)

    pools = _sc_pool(xfin_d, xfin_t, bat_d, bat_t)

    z = jnp.zeros((500, 12), f32)
    top = jnp.concatenate([adj[:500, :500], z, adj[:500, 500:], z], axis=1)
    bot = jnp.concatenate([adj[500:, :500], z, adj[500:, 500:], z], axis=1)
    zr = jnp.zeros((12, 1024), f32)
    adjp = jnp.concatenate([top, zr, bot, zr], axis=0)

    n1 = jnp.asarray(num_node1s, jnp.int32).reshape(1, 1)
    n2 = jnp.asarray(num_node2s, jnp.int32).reshape(1, 1)
    weights = (
        _pad2(do_W1, 384, 512), _padb(do_b1, 512), do_W2, _padb(do_b2, 256),
        _pad2(to_W1, 256, 512), _padb(to_b1, 512), to_W2, _padb(to_b2, 256),
        ag_W, _padb(ag_b, 256),
        dt_W1, _padb(dt_b1, 512), dt_W2, _padb(dt_b2, 128),
        tt_W1, _padb(tt_b1, 512), tt_W2, _padb(tt_b2, 128),
    )
    demb, temb = _tc_tail(pools, n1, n2, adjp, weights)
    return demb[:500], temb[:500]


# speedup vs baseline: 1.2669x; 1.2669x over previous
import functools

import jax
import jax.numpy as jnp
from jax import lax
from jax.experimental import pallas as pl
from jax.experimental.pallas import tpu as pltpu
from jax.experimental.pallas import tpu_sc as plsc

N = 10000
E = 320000
NC, NS = 2, 16
NW = NC * NS
CH = 80
CC = 128
GP = 512
RPT_A, RPT_B = 624, 640


def _mesh():
    return plsc.VectorSubcoreMesh(
        core_axis_name="c", subcore_axis_name="s", num_cores=NC,
        num_subcores=NS)


def _zero_fill(buf, rows, width):
    def row(i, _):
        for j in range(width // 16):
            buf[i, pl.ds(j * 16, 16)] = jnp.zeros((16,), jnp.float32)
        return 0
    lax.fori_loop(0, rows, row, 0, unroll=False)


def _zero_spmem_rows(zbuf, acc, base_r, nrows):
    zc = zbuf.shape[0]
    for k in range(nrows // zc):
        pltpu.sync_copy(zbuf, acc.at[pl.ds(base_r + k * zc, zc)])
    rem = nrows % zc
    if rem:
        pltpu.sync_copy(zbuf.at[pl.ds(0, rem)],
                        acc.at[pl.ds(base_r + (nrows // zc) * zc, rem)])


def _per_tile_rows(s, fn):
    @pl.when(s < NS - 1)
    def _():
        fn(s * RPT_A, RPT_A)

    @pl.when(s == NS - 1)
    def _():
        fn((NS - 1) * RPT_A, RPT_B)


def _sc_degree(dst_d, dst_t):
    nchunk = E // CC

    @functools.partial(
        pl.kernel,
        out_type=(jax.ShapeDtypeStruct((NC * N,), jnp.float32),
                  jax.ShapeDtypeStruct((NC * N,), jnp.float32)),
        mesh=_mesh(),
        scratch_types=[
            pltpu.VMEM((CC,), jnp.int32),
            pltpu.VMEM((CC,), jnp.float32),
            pltpu.VMEM((640,), jnp.float32),
            pltpu.VMEM_SHARED((N,), jnp.float32),
            pltpu.VMEM_SHARED((N,), jnp.float32),
        ],
    )
    def body(ed_hbm, et_hbm, outd_hbm, outt_hbm, dst_v, ones_v, vbuf,
             accd, acct):
        c = lax.axis_index("c")
        s = lax.axis_index("s")
        w = c * NS + s

        def fill(i, _):
            ones_v[pl.ds(i * 16, 16)] = jnp.full((16,), 1.0, jnp.float32)
            vbuf[pl.ds(i * 16, 16)] = jnp.zeros((16,), jnp.float32)
            vbuf[pl.ds(320 + i * 16, 16)] = jnp.zeros((16,), jnp.float32)
            vbuf[pl.ds(512 + i * 16, 16)] = jnp.zeros((16,), jnp.float32)
            return 0
        lax.fori_loop(0, CC // 16, fill, 0, unroll=False)

        @pl.when(s < NS - 1)
        def _():
            pltpu.sync_copy(vbuf.at[pl.ds(0, 624)],
                            accd.at[pl.ds(s * 624, 624)])
            pltpu.sync_copy(vbuf.at[pl.ds(0, 624)],
                            acct.at[pl.ds(s * 624, 624)])

        @pl.when(s == NS - 1)
        def _():
            pltpu.sync_copy(vbuf, accd.at[pl.ds((NS - 1) * 624, 640)])
            pltpu.sync_copy(vbuf, acct.at[pl.ds((NS - 1) * 624, 640)])
        plsc.subcore_barrier()

        def step(it, _):
            j = w + it * NW

            @pl.when(j < nchunk)
            def _():
                base = j * CC
                pltpu.sync_copy(ed_hbm.at[pl.ds(base, CC)], dst_v)
                pltpu.sync_copy(ones_v, accd.at[dst_v], add=True)
                pltpu.sync_copy(et_hbm.at[pl.ds(base, CC)], dst_v)
                pltpu.sync_copy(ones_v, acct.at[dst_v], add=True)
            return 0
        lax.fori_loop(0, (nchunk + NW - 1) // NW, step, 0, unroll=False)
        plsc.subcore_barrier()

        for acc, out_hbm in ((accd, outd_hbm), (acct, outt_hbm)):
            @pl.when(s < NS - 1)
            def _():
                pltpu.sync_copy(acc.at[pl.ds(s * 624, 624)],
                                vbuf.at[pl.ds(0, 624)])
                pltpu.sync_copy(vbuf.at[pl.ds(0, 624)],
                                out_hbm.at[pl.ds(c * N + s * 624, 624)])

            @pl.when(s == NS - 1)
            def _():
                pltpu.sync_copy(acc.at[pl.ds((NS - 1) * 624, 640)], vbuf)
                pltpu.sync_copy(
                    vbuf, out_hbm.at[pl.ds(c * N + (NS - 1) * 624, 640)])

    return body(dst_d, dst_t)


def _sc_conv(xs, src, dst):
    ew = E // NW
    nch = ew // CC
    tailc = ew - nch * CC

    @functools.partial(
        pl.kernel,
        out_type=jax.ShapeDtypeStruct((NC, N, 128), jnp.float32),
        mesh=_mesh(),
        scratch_types=[
            pltpu.VMEM((CC,), jnp.int32),
            pltpu.VMEM((CC,), jnp.int32),
            pltpu.VMEM((CC,), jnp.int32),
            pltpu.VMEM((CC,), jnp.int32),
            pltpu.VMEM((tailc,), jnp.int32),
            pltpu.VMEM((tailc,), jnp.int32),
            pltpu.VMEM((CC, 128), jnp.float32),
            pltpu.VMEM((CC, 128), jnp.float32),
            pltpu.VMEM((tailc, 128), jnp.float32),
            pltpu.VMEM_SHARED((N, 128), jnp.float32),
            pltpu.SemaphoreType.DMA,
            pltpu.SemaphoreType.DMA,
            pltpu.SemaphoreType.DMA,
            pltpu.SemaphoreType.DMA,
        ],
    )
    def body(xs_hbm, src_hbm, dst_hbm, out_hbm, s0v, d0v, s1v, d1v, stv,
             dtv, r0, r1, rt, acc, g0, g1, q0, q1):
        c = lax.axis_index("c")
        s = lax.axis_index("s")
        e0 = (c * NS + s) * ew
        bufs = ((s0v, d0v, r0, g0, q0), (s1v, d1v, r1, g1, q1))
        _zero_fill(r0, CC, 128)
        _per_tile_rows(s, lambda st, nr: _zero_spmem_rows(r0, acc, st, nr))
        plsc.subcore_barrier()

        def chunk(j, sv, dv, rv, gs, qs):
            base = e0 + j * CC
            pltpu.sync_copy(src_hbm.at[pl.ds(base, CC)], sv)
            pltpu.sync_copy(dst_hbm.at[pl.ds(base, CC)], dv)
            pltpu.async_copy(xs_hbm.at[sv], rv, gs).wait()
            pltpu.async_copy(rv, acc.at[dv], qs, add=True)

        def pair(i, _):
            for b, (sv, dv, rv, gs, qs) in enumerate(bufs):
                @pl.when(i >= 1)
                def _():
                    pltpu.make_async_copy(rv, acc.at[dv], qs).wait()
                chunk(2 * i + b, sv, dv, rv, gs, qs)
            return 0
        lax.fori_loop(0, nch // 2, pair, 0, unroll=False)
        pltpu.make_async_copy(r0, acc.at[d0v], q0).wait()
        pltpu.make_async_copy(r1, acc.at[d1v], q1).wait()
        base = e0 + nch * CC
        pltpu.sync_copy(src_hbm.at[pl.ds(base, tailc)], stv)
        pltpu.sync_copy(dst_hbm.at[pl.ds(base, tailc)], dtv)
        pltpu.async_copy(xs_hbm.at[stv], rt, g0).wait()
        pltpu.sync_copy(rt, acc.at[dtv], add=True)
        plsc.subcore_barrier()

        def wb(start, nrows):
            pltpu.sync_copy(acc.at[pl.ds(start, nrows)],
                            out_hbm.at[c, pl.ds(start, nrows)])
        _per_tile_rows(s, wb)

    return body(xs, src, dst)


def _sc_conv_cat(xs2, src, dst):
    ew = E // NS
    nch = ew // CC
    tailc = ew - nch * CC

    @functools.partial(
        pl.kernel,
        out_type=jax.ShapeDtypeStruct((NC, N, 128), jnp.float32),
        mesh=_mesh(),
        scratch_types=[
            pltpu.VMEM((CC,), jnp.int32),
            pltpu.VMEM((CC,), jnp.int32),
            pltpu.VMEM((CC,), jnp.int32),
            pltpu.VMEM((CC,), jnp.int32),
            pltpu.VMEM((tailc,), jnp.int32),
            pltpu.VMEM((tailc,), jnp.int32),
            pltpu.VMEM((CC, 128), jnp.float32),
            pltpu.VMEM((CC, 128), jnp.float32),
            pltpu.VMEM((tailc, 128), jnp.float32),
            pltpu.VMEM_SHARED((N, 128), jnp.float32),
            pltpu.SemaphoreType.DMA,
            pltpu.SemaphoreType.DMA,
            pltpu.SemaphoreType.DMA,
            pltpu.SemaphoreType.DMA,
        ],
    )
    def body(xs_hbm, src_hbm, dst_hbm, out_hbm, s0v, d0v, s1v, d1v, stv,
             dtv, r0, r1, rt, acc, g0, g1, q0, q1):
        c = lax.axis_index("c")
        s = lax.axis_index("s")
        e0 = s * ew
        off = c * N
        bufs = ((s0v, d0v, r0, g0, q0), (s1v, d1v, r1, g1, q1))
        _zero_fill(r0, CC, 128)
        _per_tile_rows(s, lambda st, nr: _zero_spmem_rows(r0, acc, st, nr))
        plsc.subcore_barrier()

        def chunk(j, sv, dv, rv, gs, qs):
            base = e0 + j * CC
            pltpu.sync_copy(src_hbm.at[pl.ds(base, CC)], sv)
            pltpu.sync_copy(dst_hbm.at[pl.ds(base, CC)], dv)
            for k in range(CC // 16):
                sl = pl.ds(k * 16, 16)
                sv[sl] = sv[sl] + off
            pltpu.async_copy(xs_hbm.at[sv], rv, gs).wait()
            pltpu.async_copy(rv, acc.at[dv], qs, add=True)

        def pair(i, _):
            for b, (sv, dv, rv, gs, qs) in enumerate(bufs):
                @pl.when(i >= 1)
                def _():
                    pltpu.make_async_copy(rv, acc.at[dv], qs).wait()
                chunk(2 * i + b, sv, dv, rv, gs, qs)
            return 0
        lax.fori_loop(0, nch // 2, pair, 0, unroll=False)
        pltpu.make_async_copy(r0, acc.at[d0v], q0).wait()
        pltpu.make_async_copy(r1, acc.at[d1v], q1).wait()
        base = e0 + nch * CC
        pltpu.sync_copy(src_hbm.at[pl.ds(base, tailc)], stv)
        pltpu.sync_copy(dst_hbm.at[pl.ds(base, tailc)], dtv)
        for k in range(tailc // 16):
            sl = pl.ds(k * 16, 16)
            stv[sl] = stv[sl] + off
        pltpu.async_copy(xs_hbm.at[stv], rt, g0).wait()
        pltpu.sync_copy(rt, acc.at[dtv], add=True)
        plsc.subcore_barrier()

        def wb(start, nrows):
            pltpu.sync_copy(acc.at[pl.ds(start, nrows)],
                            out_hbm.at[c, pl.ds(start, nrows)])
        _per_tile_rows(s, wb)

    return body(xs2, src, dst)


def _sc_pool(xd, xt, batch_d, batch_t):
    rows_g = GP // NS
    o128 = jax.ShapeDtypeStruct((NC, GP, 128), jnp.float32)

    @functools.partial(
        pl.kernel,
        out_type=(o128,) * 7,
        mesh=_mesh(),
        scratch_types=[
            pltpu.VMEM((CH,), jnp.int32),
            pltpu.VMEM((CH, 128), jnp.float32),
            pltpu.VMEM((CH, 128), jnp.float32),
            pltpu.VMEM((CH, 128), jnp.float32),
            pltpu.VMEM((CH, 128), jnp.float32),
            pltpu.VMEM_SHARED((GP, 128), jnp.float32),
            pltpu.VMEM_SHARED((GP, 128), jnp.float32),
            pltpu.VMEM_SHARED((GP, 128), jnp.float32),
            pltpu.VMEM_SHARED((GP, 128), jnp.float32),
            pltpu.VMEM_SHARED((GP, 128), jnp.float32),
            pltpu.VMEM_SHARED((GP, 128), jnp.float32),
            pltpu.VMEM_SHARED((GP, 128), jnp.float32),
        ],
    )
    def body(xd_hbm, xt_hbm, bd_hbm, bt_hbm,
             pd0_h, pd1_h, pd2_h, pt0_h, pt1_h, cd_h, ct_h,
             bidx, ra, rb, rc, ones_v,
             ad0, ad1, ad2, at0, at1, cntd, cntt):
        c = lax.axis_index("c")
        s = lax.axis_index("s")
        w = c * NS + s
        base_g = s * rows_g
        _zero_fill(ra, rows_g, 128)
        for accr in (ad0, ad1, ad2, at0, at1, cntd, cntt):
            pltpu.sync_copy(ra.at[pl.ds(0, rows_g)],
                            accr.at[pl.ds(base_g, rows_g)])

        def fill1(i, _):
            for j in range(8):
                ones_v[i, pl.ds(j * 16, 16)] = jnp.full((16,), 1.0,
                                                        jnp.float32)
            return 0
        lax.fori_loop(0, CH, fill1, 0, unroll=False)
        plsc.subcore_barrier()

        nchunk = N // CH

        def step(it, _):
            j = w + it * NW

            @pl.when(j < nchunk)
            def _():
                base = j * CH
                pltpu.sync_copy(bd_hbm.at[pl.ds(base, CH)], bidx)
                for k, (rbuf, accr) in enumerate(
                        ((ra, ad0), (rb, ad1), (rc, ad2))):
                    pltpu.sync_copy(
                        xd_hbm.at[pl.ds(base, CH), pl.ds(k * 128, 128)],
                        rbuf)
                    pltpu.sync_copy(rbuf, accr.at[bidx], add=True)
                pltpu.sync_copy(ones_v, cntd.at[bidx], add=True)
                pltpu.sync_copy(bt_hbm.at[pl.ds(base, CH)], bidx)
                for k, (rbuf, accr) in enumerate(((ra, at0), (rb, at1))):
                    pltpu.sync_copy(
                        xt_hbm.at[pl.ds(base, CH), pl.ds(k * 128, 128)],
                        rbuf)
                    pltpu.sync_copy(rbuf, accr.at[bidx], add=True)
                pltpu.sync_copy(ones_v, cntt.at[bidx], add=True)
            return 0
        lax.fori_loop(0, (nchunk + NW - 1) // NW, step, 0, unroll=False)
        plsc.subcore_barrier()
        for accr, out_h in ((ad0, pd0_h), (ad1, pd1_h), (ad2, pd2_h),
                            (at0, pt0_h), (at1, pt1_h), (cntd, cd_h),
                            (cntt, ct_h)):
            pltpu.sync_copy(accr.at[pl.ds(base_g, rows_g)],
                            out_h.at[c, pl.ds(base_g, rows_g)])

    return body(xd, xt, batch_d, batch_t)


_BR = 1000


def _tc_prep(cnt_d, cnt_t, xd, xt):
    def body(cd_r, ct_r, xd_r, xt_r, dd_r, dt_r, od_r, ot_r):
        degd = jnp.sum(cd_r[0], axis=1)[:, None] + 1.0
        dd = lax.rsqrt(jnp.broadcast_to(degd, (_BR, 8)))
        dd_r[...] = dd
        od_r[...] = xd_r[...] * dd[:, 0:1]
        degt = jnp.sum(ct_r[0], axis=1)[:, None] + 1.0
        dt = lax.rsqrt(jnp.broadcast_to(degt, (_BR, 8)))
        dt_r[...] = dt
        ot_r[...] = xt_r[...] * dt[:, 0:1]

    grid = N // _BR
    return pl.pallas_call(
        body,
        grid=(grid,),
        in_specs=[
            pl.BlockSpec((1, _BR, NC), lambda i: (i, 0, 0)),
            pl.BlockSpec((1, _BR, NC), lambda i: (i, 0, 0)),
            pl.BlockSpec((_BR, 128), lambda i: (i, 0)),
            pl.BlockSpec((_BR, 128), lambda i: (i, 0)),
        ],
        out_specs=[
            pl.BlockSpec((_BR, 8), lambda i: (i, 0)),
            pl.BlockSpec((_BR, 8), lambda i: (i, 0)),
            pl.BlockSpec((_BR, 128), lambda i: (i, 0)),
            pl.BlockSpec((_BR, 128), lambda i: (i, 0)),
        ],
        out_shape=[
            jax.ShapeDtypeStruct((N, 8), jnp.float32),
            jax.ShapeDtypeStruct((N, 8), jnp.float32),
            jax.ShapeDtypeStruct((N, 128), jnp.float32),
            jax.ShapeDtypeStruct((N, 128), jnp.float32),
        ],
    )(cnt_d, cnt_t, xd, xt)


def _mm(a, b):
    return lax.dot_general(a, b, (((1,), (0,)), ((), ())),
                           precision=lax.Precision.HIGHEST,
                           preferred_element_type=jnp.float32)


def _tc_layer(acc, xs, dis, wp, bp, scale_end, cat_in=False,
              halves_out=False):
    win, wout = wp.shape

    def body(acc_r, xs_r, dis_r, w_r, b_r, out_r):
        if cat_in:
            a = jnp.concatenate([acc_r[0] + xs_r[0], acc_r[1] + xs_r[1]],
                                axis=1)
        else:
            a = acc_r[0] + acc_r[1] + xs_r[...]
        a = a * dis_r[:, 0:1]
        h = _mm(a, w_r[...])
        h = jnp.maximum(h + b_r[...], 0.0)
        if scale_end:
            h = h * dis_r[:, 0:1]
        if halves_out:
            out_r[0] = h[:, 0:128]
            out_r[1] = h[:, 128:256]
        else:
            out_r[...] = h

    grid = N // _BR
    if cat_in:
        in0 = pl.BlockSpec((NC, _BR, 128), lambda i: (0, i, 0))
        in1 = pl.BlockSpec((NC, _BR, 128), lambda i: (0, i, 0))
    else:
        in0 = pl.BlockSpec((NC, _BR, win), lambda i: (0, i, 0))
        in1 = pl.BlockSpec((_BR, win), lambda i: (i, 0))
    if halves_out:
        out_spec = pl.BlockSpec((NC, _BR, 128), lambda i: (0, i, 0))
        out_shape = jax.ShapeDtypeStruct((NC, N, 128), jnp.float32)
    else:
        out_spec = pl.BlockSpec((_BR, wout), lambda i: (i, 0))
        out_shape = jax.ShapeDtypeStruct((N, wout), jnp.float32)
    return pl.pallas_call(
        body,
        grid=(grid,),
        in_specs=[
            in0, in1,
            pl.BlockSpec((_BR, 8), lambda i: (i, 0)),
            pl.BlockSpec((win, wout), lambda i: (0, 0)),
            pl.BlockSpec((1, wout), lambda i: (0, 0)),
        ],
        out_specs=out_spec,
        out_shape=out_shape,
    )(acc, xs, dis, wp, bp)


def _tc_tail(pools, n1, n2, adjp, weights):
    def body(pd0_r, pd1_r, pd2_r, pt0_r, pt1_r, cd_r, ct_r, n1_r, n2_r,
             adj_r,
             dow1_r, dob1_r, dow2_r, dob2_r, tow1_r, tob1_r, tow2_r, tob2_r,
             agw_r, agb_r, dtw1_r, dtb1_r, dtw2_r, dtb2_r,
             ttw1_r, ttb1_r, ttw2_r, ttb2_r, demb_r, temb_r):
        iota = lax.broadcasted_iota(jnp.int32, (GP, 1), 0)
        psum_d = jnp.concatenate(
            [pd0_r[0] + pd0_r[1], pd1_r[0] + pd1_r[1], pd2_r[0] + pd2_r[1]],
            axis=1)
        psum_t = jnp.concatenate(
            [pt0_r[0] + pt0_r[1], pt1_r[0] + pt1_r[1]], axis=1)
        dge = psum_d / jnp.maximum((cd_r[0] + cd_r[1])[:, 0:1], 1.0)
        dge = jnp.where(iota < n1_r[0, 0], dge, 0.0)
        tge = psum_t / jnp.maximum((ct_r[0] + ct_r[1])[:, 0:1], 1.0)
        tge = jnp.where(iota < n2_r[0, 0], tge, 0.0)
        dout = _mm(jnp.maximum(_mm(dge, dow1_r[...]) + dob1_r[...], 0.0),
                   dow2_r[...]) + dob2_r[...]
        tout = _mm(jnp.maximum(_mm(tge, tow1_r[...]) + tob1_r[...], 0.0),
                   tow2_r[...]) + tob2_r[...]
        feat = jnp.concatenate([dout, tout], axis=0)
        deg = jnp.clip(jnp.sum(adj_r[...], axis=1, keepdims=True), 1.0, None)
        disa = lax.rsqrt(deg)
        h = _mm(feat, agw_r[...])
        aff = jnp.maximum(disa * _mm(adj_r[...], disa * h) + agb_r[...], 0.0)
        dtr = _mm(jnp.maximum(_mm(aff[0:GP], dtw1_r[...]) + dtb1_r[...],
                              0.0), dtw2_r[...]) + dtb2_r[...]
        ttr = _mm(jnp.maximum(
            _mm(aff[GP:2 * GP], ttw1_r[...]) + ttb1_r[...], 0.0),
            ttw2_r[...]) + ttb2_r[...]
        demb_r[:, 0:256] = dout
        demb_r[:, 256:384] = dtr
        temb_r[:, 0:256] = tout
        temb_r[:, 256:384] = ttr

    return pl.pallas_call(
        body,
        out_shape=[
            jax.ShapeDtypeStruct((GP, 384), jnp.float32),
            jax.ShapeDtypeStruct((GP, 384), jnp.float32),
        ],
    )(*pools, n1, n2, adjp, *weights)


def _pad2(a, r, c):
    return jnp.pad(a, ((0, r - a.shape[0]), (0, c - a.shape[1])))


def _padb(b, n):
    return jnp.pad(b, (0, n - b.shape[0])).reshape(1, n)


def kernel(x_drug, edge_index_drug, batch_drug, x_target, edge_index_target,
           batch_target, adj, num_node1s, num_node2s, dg_W1, dg_b1, dg_W2,
           dg_b2, dg_W3, dg_b3, tg_W1, tg_b1, tg_W2, tg_b2, tg_W3, tg_b3,
           do_W1, do_b1, do_W2, do_b2, to_W1, to_b1, to_W2, to_b2, ag_W,
           ag_b, dt_W1, dt_b1, dt_W2, dt_b2, tt_W1, tt_b1, tt_W2, tt_b2):
    f32 = jnp.float32
    src_d = edge_index_drug[0].astype(jnp.int32)
    dst_d = edge_index_drug[1].astype(jnp.int32)
    src_t = edge_index_target[0].astype(jnp.int32)
    dst_t = edge_index_target[1].astype(jnp.int32)
    bat_d = batch_drug.astype(jnp.int32)
    bat_t = batch_target.astype(jnp.int32)

    xd = _pad2(x_drug.astype(f32), N, 128)
    xt = _pad2(x_target.astype(f32), N, 128)

    cnt_d, cnt_t = _sc_degree(dst_d, dst_t)
    cnt_d = cnt_d.reshape(NC, N // _BR, _BR).transpose(1, 2, 0)
    cnt_t = cnt_t.reshape(NC, N // _BR, _BR).transpose(1, 2, 0)
    dis_d, dis_t, xs_d, xs_t = _tc_prep(cnt_d, cnt_t, xd, xt)

    acc = _sc_conv(xs_d, src_d, dst_d)
    xs_d = _tc_layer(acc, xs_d, dis_d, _pad2(dg_W1, 128, 128),
                     _padb(dg_b1, 128), True)
    acc = _sc_conv(xs_d, src_d, dst_d)
    xs3_d = _tc_layer(acc, xs_d, dis_d, _pad2(dg_W2, 128, 256),
                      _padb(dg_b2, 256), True, halves_out=True)
    acc = _sc_conv_cat(xs3_d.reshape(NC * N, 128), src_d, dst_d)
    xfin_d = _tc_layer(acc, xs3_d, dis_d, _pad2(dg_W3, 256, 384),
                       _padb(dg_b3, 384), False, cat_in=True)

    acc = _sc_conv(xs_t, src_t, dst_t)
    xs_t = _tc_layer(acc, xs_t, dis_t, _pad2(tg_W1, 128, 128),
                     _padb(tg_b1, 128), True)
    acc = _sc_conv(xs_t, src_t, dst_t)
    xs_t = _tc_layer(acc, xs_t, dis_t, _pad2(tg_W2, 128, 128),
                     _padb(tg_b2, 128), True)
    acc = _sc_conv(xs_t, src_t, dst_t)
    xfin_t = _tc_layer(acc, xs_t, dis_t, _pad2(tg_W3, 128, 256),
                       _padb(tg_b3, 256), False)

    pools = _sc_pool(xfin_d, xfin_t, bat_d, bat_t)

    z = jnp.zeros((500, 12), f32)
    top = jnp.concatenate([adj[:500, :500], z, adj[:500, 500:], z], axis=1)
    bot = jnp.concatenate([adj[500:, :500], z, adj[500:, 500:], z], axis=1)
    zr = jnp.zeros((12, 1024), f32)
    adjp = jnp.concatenate([top, zr, bot, zr], axis=0)

    n1 = jnp.asarray(num_node1s, jnp.int32).reshape(1, 1)
    n2 = jnp.asarray(num_node2s, jnp.int32).reshape(1, 1)
    weights = (
        _pad2(do_W1, 384, 512), _padb(do_b1, 512), do_W2, _padb(do_b2, 256),
        _pad2(to_W1, 256, 512), _padb(to_b1, 512), to_W2, _padb(to_b2, 256),
        ag_W, _padb(ag_b, 256),
        dt_W1, _padb(dt_b1, 512), dt_W2, _padb(dt_b2, 128),
        tt_W1, _padb(tt_b1, 512), tt_W2, _padb(tt_b2, 128),
    )
    demb, temb = _tc_tail(pools, n1, n2, adjp, weights)
    return demb[:500], temb[:500]
